# Initial kernel scaffold; baseline (speedup 1.0000x reference)
import jax
import jax.numpy as jnp
from jax.experimental import pallas as pl


def kernel(center_words, pos_context, neg_context, emb_v, emb_u):
    raise NotImplementedError("write your pallas kernel here")


# speedup vs baseline: 4.1590x; 4.1590x over previous
import functools

import jax
import jax.numpy as jnp
from jax import lax
from jax.experimental import pallas as pl
from jax.experimental.pallas import tpu as pltpu
from jax.experimental.pallas import tpu_sc as plsc

VOCAB = 1000000
DIM = 64
B = 16384
K_NEG = 20

NC = 2
NS = 16
NW = NC * NS
BPW = B // NW
IDX_W = 128
NEG_ROWS = BPW * K_NEG
NEG_J = NEG_ROWS // IDX_W
NEG_CHUNK = 8
ROWS_BUF = NEG_CHUNK * IDX_W


def _gather_body(cw, pc, nc, vtab, utab, outc, outp, outn, cidx, pidx, nidx,
                 rows, sem):
    c = lax.axis_index("c")
    s = lax.axis_index("s")
    wid = s * NC + c
    base = wid * BPW

    pltpu.sync_copy(cw.at[wid], cidx)
    pltpu.sync_copy(pc.at[wid], pidx)
    pltpu.sync_copy(nc.at[wid], nidx)

    hs = [pltpu.async_copy(vtab.at[cidx.at[j]], rows.at[pl.ds(j * IDX_W, IDX_W)], sem)
          for j in range(BPW // IDX_W)]
    for h in hs:
        h.wait()
    pltpu.sync_copy(rows.at[pl.ds(0, BPW)], outc.at[pl.ds(base, BPW)])

    hs = [pltpu.async_copy(utab.at[pidx.at[j]], rows.at[pl.ds(j * IDX_W, IDX_W)], sem)
          for j in range(BPW // IDX_W)]
    for h in hs:
        h.wait()
    pltpu.sync_copy(rows.at[pl.ds(0, BPW)], outp.at[pl.ds(base, BPW)])

    for chunk in range(NEG_J // NEG_CHUNK):
        hs = [pltpu.async_copy(utab.at[nidx.at[chunk * NEG_CHUNK + j]],
                               rows.at[pl.ds(j * IDX_W, IDX_W)], sem)
              for j in range(NEG_CHUNK)]
        for h in hs:
            h.wait()
        pltpu.sync_copy(rows, outn.at[pl.ds(wid * NEG_ROWS + chunk * ROWS_BUF,
                                            ROWS_BUF)])


@functools.cache
def _gather_rows():
    return pl.kernel(
        _gather_body,
        out_type=(
            jax.ShapeDtypeStruct((B, DIM), jnp.float32),
            jax.ShapeDtypeStruct((B, DIM), jnp.float32),
            jax.ShapeDtypeStruct((B * K_NEG, DIM), jnp.float32),
        ),
        mesh=plsc.VectorSubcoreMesh(core_axis_name="c", subcore_axis_name="s"),
        scratch_types=(
            pltpu.VMEM((BPW // IDX_W, IDX_W), jnp.int32),
            pltpu.VMEM((BPW // IDX_W, IDX_W), jnp.int32),
            pltpu.VMEM((NEG_J, IDX_W), jnp.int32),
            pltpu.VMEM((ROWS_BUF, DIM), jnp.float32),
            pltpu.SemaphoreType.DMA,
        ),
        compiler_params=pltpu.CompilerParams(use_tc_tiling_on_sc=False),
    )


BLK = 2048


def _loss_body(c_ref, p_ref, n_ref, o_ref):
    i = pl.program_id(0)
    c = c_ref[...]
    p = p_ref[...]
    n = n_ref[...]
    pos_dot = jnp.sum(c * p, axis=-1)
    neg_dot = jnp.sum(n * c[:, None, :], axis=-1)
    part = (jnp.sum(jax.nn.log_sigmoid(pos_dot))
            + jnp.sum(jax.nn.log_sigmoid(-neg_dot)))

    @pl.when(i == 0)
    def _():
        o_ref[0, 0] = 0.0

    o_ref[0, 0] += -part


def _loss(center_embed, pos_embed, neg_embed):
    out = pl.pallas_call(
        _loss_body,
        grid=(B // BLK,),
        in_specs=[
            pl.BlockSpec((BLK, DIM), lambda i: (i, 0)),
            pl.BlockSpec((BLK, DIM), lambda i: (i, 0)),
            pl.BlockSpec((BLK, K_NEG, DIM), lambda i: (i, 0, 0)),
        ],
        out_specs=pl.BlockSpec(memory_space=pltpu.SMEM),
        out_shape=jax.ShapeDtypeStruct((1, 1), jnp.float32),
    )(center_embed, pos_embed, neg_embed)
    return out[0, 0]


def kernel(center_words, pos_context, neg_context, emb_v, emb_u):
    cw = center_words.astype(jnp.int32).reshape(NW, BPW // IDX_W, IDX_W)
    pc = pos_context.astype(jnp.int32).reshape(NW, BPW // IDX_W, IDX_W)
    nc = neg_context.astype(jnp.int32).reshape(NW, NEG_J, IDX_W)
    center_embed, pos_embed, neg_flat = _gather_rows()(cw, pc, nc, emb_v, emb_u)
    neg_embed = neg_flat.reshape(B, K_NEG, DIM)
    return _loss(center_embed, pos_embed, neg_embed)
